# Initial kernel scaffold; baseline (speedup 1.0000x reference)
import jax
import jax.numpy as jnp
from jax.experimental import pallas as pl


def kernel(x, edge_attr, W1, b1, W2, b2, W3, b3, edge_index, k):
    raise NotImplementedError("write your pallas kernel here")


# speedup vs baseline: 27.6229x; 27.6229x over previous
import functools

import jax
import jax.numpy as jnp
from jax import lax
from jax.experimental import pallas as pl
from jax.experimental.pallas import tpu as pltpu
from jax.experimental.pallas import tpu_sc as plsc

N_NODES = 100000
N_PAD = 102400
E_EDGES = 3200000
E_PAD = 3211264
ROWS_E = E_PAD // 128
NC, NS = 2, 16
NW = NC * NS
RW = ROWS_E // NW
U = 4
NCHUNK = RW // U
SLICE = N_PAD // NS

_mesh = lambda: plsc.VectorSubcoreMesh(
    core_axis_name="c", subcore_axis_name="s", num_cores=NC, num_subcores=NS)


def _stage_plane(src_at, dst_sp_at, va):
    pltpu.sync_copy(src_at, va)
    pltpu.sync_copy(va, dst_sp_at)


def _vec_loop(n, body):
    lax.fori_loop(0, n, lambda i, _: (body(i), 0)[1], 0)


def _zero_buf(buf, n):
    def b(i):
        buf[pl.ds(i * 16, 16)] = jnp.zeros((16,), jnp.float32)
    _vec_loop(n // 16, b)


def _make_deg_kernel():
    @functools.partial(
        pl.kernel,
        out_type=jax.ShapeDtypeStruct((NC, N_PAD), jnp.float32),
        mesh=_mesh(),
        scratch_types=[
            pltpu.VMEM_SHARED((N_PAD,), jnp.float32),
            pltpu.VMEM((SLICE,), jnp.float32),
            pltpu.VMEM((U, 128), jnp.int32),
            pltpu.VMEM((U, 128), jnp.float32),
        ],
    )
    def deg_kernel(col_hbm, ew_hbm, out_hbm, acc, zbuf, colv, ewv):
        cid = lax.axis_index("c")
        sid = lax.axis_index("s")
        wid = cid * NS + sid
        base = sid * SLICE
        _zero_buf(zbuf, SLICE)
        pltpu.sync_copy(zbuf, acc.at[pl.ds(base, SLICE)])
        plsc.subcore_barrier()
        erow0 = wid * RW

        def chunk(ci, _):
            r0 = erow0 + ci * U
            pltpu.sync_copy(col_hbm.at[pl.ds(r0, U)], colv)
            pltpu.sync_copy(ew_hbm.at[pl.ds(r0, U)], ewv)
            for j in range(U):
                for v in range(8):
                    s = pl.ds(v * 16, 16)
                    ewv[j, s] = jnp.abs(ewv[j, s])
            for j in range(U):
                pltpu.sync_copy(ewv.at[j], acc.at[colv.at[j]], add=True)
            return 0

        lax.fori_loop(0, NCHUNK, chunk, 0)
        plsc.subcore_barrier()
        pltpu.sync_copy(acc.at[pl.ds(base, SLICE)],
                        out_hbm.at[cid, pl.ds(base, SLICE)])

    return deg_kernel


def _make_norm_kernel():
    @functools.partial(
        pl.kernel,
        out_type=jax.ShapeDtypeStruct((ROWS_E, 128), jnp.float32),
        mesh=_mesh(),
        scratch_types=[
            pltpu.VMEM_SHARED((N_PAD,), jnp.float32),
            pltpu.VMEM((SLICE,), jnp.float32),
            pltpu.VMEM((U, 128), jnp.int32),
            pltpu.VMEM((U, 128), jnp.int32),
            pltpu.VMEM((U, 128), jnp.float32),
            pltpu.VMEM((U, 128), jnp.float32),
            pltpu.VMEM((U, 128), jnp.float32),
        ],
    )
    def norm_kernel(dis_hbm, row_hbm, col_hbm, ew_hbm, out_hbm,
                    dis_sp, va, rowv, colv, ewv, gr, gc):
        cid = lax.axis_index("c")
        sid = lax.axis_index("s")
        wid = cid * NS + sid
        base = sid * SLICE
        _stage_plane(dis_hbm.at[pl.ds(base, SLICE)],
                     dis_sp.at[pl.ds(base, SLICE)], va)
        plsc.subcore_barrier()
        erow0 = wid * RW

        def chunk(ci, _):
            r0 = erow0 + ci * U
            pltpu.sync_copy(row_hbm.at[pl.ds(r0, U)], rowv)
            pltpu.sync_copy(col_hbm.at[pl.ds(r0, U)], colv)
            pltpu.sync_copy(ew_hbm.at[pl.ds(r0, U)], ewv)
            for j in range(U):
                pltpu.sync_copy(dis_sp.at[rowv.at[j]], gr.at[j])
                pltpu.sync_copy(dis_sp.at[colv.at[j]], gc.at[j])
            for j in range(U):
                for v in range(8):
                    s = pl.ds(v * 16, 16)
                    ewv[j, s] = gr[j, s] * jnp.abs(ewv[j, s]) * gc[j, s]
            pltpu.sync_copy(ewv, out_hbm.at[pl.ds(r0, U)])
            return 0

        lax.fori_loop(0, NCHUNK, chunk, 0)

    return norm_kernel


def _make_prop_kernel(P, n_in):
    scratch = (
        [pltpu.VMEM_SHARED((N_PAD,), jnp.float32) for _ in range(P)]
        + [pltpu.VMEM_SHARED((N_PAD,), jnp.float32) for _ in range(P)]
        + [pltpu.VMEM((SLICE,), jnp.float32),
           pltpu.VMEM((SLICE,), jnp.float32),
           pltpu.VMEM((SLICE,), jnp.float32),
           pltpu.VMEM((U, 128), jnp.int32),
           pltpu.VMEM((U, 128), jnp.int32),
           pltpu.VMEM((U, 128), jnp.float32),
           pltpu.VMEM((U, 128), jnp.float32)]
    )

    @functools.partial(
        pl.kernel,
        out_type=jax.ShapeDtypeStruct((NC, P, N_PAD), jnp.float32),
        mesh=_mesh(),
        scratch_types=scratch,
    )
    def prop_kernel(*refs):
        ins = refs[:n_in]
        row_hbm, col_hbm, norm_hbm, out_hbm = refs[n_in:n_in + 4]
        sc = refs[n_in + 4:]
        h_sp = sc[:P]
        acc_sp = sc[P:2 * P]
        va, vb, zbuf, rowv, colv, normv, gath = sc[2 * P:]

        cid = lax.axis_index("c")
        sid = lax.axis_index("s")
        wid = cid * NS + sid
        base = sid * SLICE
        _zero_buf(zbuf, SLICE)
        for p in range(P):
            pltpu.sync_copy(ins[0].at[p, pl.ds(base, SLICE)], va)
            for a in range(1, n_in):
                pltpu.sync_copy(ins[a].at[p, pl.ds(base, SLICE)], vb)

                def addb(i):
                    s = pl.ds(i * 16, 16)
                    va[s] = va[s] + vb[s]
                _vec_loop(SLICE // 16, addb)
            pltpu.sync_copy(va, h_sp[p].at[pl.ds(base, SLICE)])
            pltpu.sync_copy(zbuf, acc_sp[p].at[pl.ds(base, SLICE)])
        plsc.subcore_barrier()
        erow0 = wid * RW

        def chunk(ci, _):
            r0 = erow0 + ci * U
            pltpu.sync_copy(row_hbm.at[pl.ds(r0, U)], rowv)
            pltpu.sync_copy(col_hbm.at[pl.ds(r0, U)], colv)
            pltpu.sync_copy(norm_hbm.at[pl.ds(r0, U)], normv)
            for p in range(P):
                for j in range(U):
                    pltpu.sync_copy(h_sp[p].at[rowv.at[j]], gath.at[j])
                    for v in range(8):
                        s = pl.ds(v * 16, 16)
                        gath[j, s] = gath[j, s] * normv[j, s]
                    pltpu.sync_copy(gath.at[j], acc_sp[p].at[colv.at[j]],
                                    add=True)
            return 0

        lax.fori_loop(0, NCHUNK, chunk, 0)
        plsc.subcore_barrier()
        for p in range(P):
            pltpu.sync_copy(acc_sp[p].at[pl.ds(base, SLICE)],
                            out_hbm.at[cid, p, pl.ds(base, SLICE)])

    return prop_kernel


def _tc_dis(dega, degb):
    def body(a_ref, b_ref, o_ref):
        d = a_ref[...] + b_ref[...]
        o_ref[...] = jnp.where(d > 0.0, lax.rsqrt(d), 0.0)
    return pl.pallas_call(
        body, out_shape=jax.ShapeDtypeStruct((800, 128), jnp.float32),
    )(dega, degb)


def _tc_layer1(x2, s1a, s1b, s2a, s2b, s3a, s3b, w, b):
    def body(x_ref, a1, b1r, a2, b2r, a3, b3r, w_ref, bias_ref, o_ref):
        s0 = x_ref[...]
        s1 = a1[...] + b1r[...]
        s2 = a2[...] + b2r[...]
        s3 = a3[...] + b3r[...]
        planes = []
        for c in range(8):
            acc = (s0 * w_ref[0, c] + s1 * w_ref[1, c]
                   + s2 * w_ref[2, c] + s3 * w_ref[3, c] + bias_ref[0, c])
            planes.append(jnp.maximum(acc, 0.0))
        o_ref[...] = jnp.stack(planes, axis=0)
    smem = pl.BlockSpec(memory_space=pltpu.SMEM)
    return pl.pallas_call(
        body,
        out_shape=jax.ShapeDtypeStruct((8, 800, 128), jnp.float32),
        in_specs=[pl.BlockSpec()] * 7 + [smem, smem],
    )(x2, s1a, s1b, s2a, s2b, s3a, s3b, w, b)


def _tc_layer2(h1, g1a, g1b, g2a, g2b, g3a, g3b, w2, b2, w3):
    GB = 200

    def body(h_ref, a1, b1r, a2, b2r, a3, b3r, w2_ref, bias_ref, w3_ref, o_ref):
        bf = lambda v: v.astype(jnp.bfloat16).astype(jnp.float32)
        h = bf(h_ref[...])
        g1 = bf(a1[...] + b1r[...])
        g2 = bf(a2[...] + b2r[...])
        g3 = bf(a3[...] + b3r[...])
        h2 = []
        for c in range(8):
            acc = bias_ref[0, c]
            accv = jnp.full((GB, 128), acc, jnp.float32)
            for d in range(8):
                accv = accv + h[d] * bf(w2_ref[d, c])
                accv = accv + g1[d] * bf(w2_ref[8 + d, c])
                accv = accv + g2[d] * bf(w2_ref[16 + d, c])
                accv = accv + g3[d] * bf(w2_ref[24 + d, c])
            h2.append(bf(jnp.maximum(accv, 0.0)))
        zs = []
        for i in range(4):
            z = h2[0] * bf(w3_ref[i, 0])
            for c in range(1, 8):
                z = z + h2[c] * bf(w3_ref[i, c])
            zs.append(z)
        o_ref[...] = jnp.stack(zs, axis=0)

    smem = pl.BlockSpec(memory_space=pltpu.SMEM)
    big = pl.BlockSpec((8, GB, 128), lambda r: (0, r, 0))
    return pl.pallas_call(
        body,
        grid=(800 // GB,),
        out_shape=jax.ShapeDtypeStruct((4, 800, 128), jnp.float32),
        in_specs=[big] * 7 + [smem, smem, smem],
        out_specs=pl.BlockSpec((4, GB, 128), lambda r: (0, r, 0)),
    )(h1, g1a, g1b, g2a, g2b, g3a, g3b, w2, b2, w3)


def _tc_final(p3a, p3b, z0, b3, karr):
    def body(a_ref, b_ref, z_ref, b3_ref, k_ref, xs_ref, top_ref):
        INT_MIN = jnp.int32(-(2 ** 31))
        NF = jnp.float32(N_NODES)
        pre = a_ref[...] + b_ref[...] + z_ref[...] + b3_ref[0, 0]
        h3 = jnp.maximum(pre, 0.0)
        ri = lax.broadcasted_iota(jnp.int32, (800, 128), 0)
        ci = lax.broadcasted_iota(jnp.int32, (800, 128), 1)
        fidx = ri * 128 + ci
        mask = fidx < N_NODES
        h3m = jnp.where(mask, h3, 0.0)
        mean = jnp.sum(h3m) / NF
        dev = jnp.where(mask, h3 - mean, 0.0)
        var = jnp.sum(dev * dev) / NF
        xs = (h3 - mean) * lax.rsqrt(var + 1e-5)
        xs_ref[...] = xs
        u = lax.bitcast_convert_type(xs, jnp.int32)
        key = u ^ (jnp.right_shift(u, 31) & jnp.int32(0x7FFFFFFF))
        key = jnp.where(mask, key, INT_MIN)
        kf = k_ref[0, 0].astype(jnp.float32)
        cnt0 = jnp.sum(jnp.where(key >= 0, 1.0, 0.0))
        T = jnp.where(cnt0 >= kf, jnp.int32(0), INT_MIN)
        for bit in range(30, -1, -1):
            cand = T + jnp.int32(1 << bit)
            cnt = jnp.sum(jnp.where(key >= cand, 1.0, 0.0))
            T = jnp.where(cnt >= kf, cand, T)
        cnt_gt = jnp.sum(jnp.where(key > T, 1.0, 0.0))
        needed = kf - cnt_gt
        tie = jnp.where(key == T, 1.0, 0.0)
        ia = lax.broadcasted_iota(jnp.int32, (128, 128), 0)
        ib = lax.broadcasted_iota(jnp.int32, (128, 128), 1)
        m = (ia < ib).astype(jnp.float32)
        pref_in_row = jnp.dot(tie, m, preferred_element_type=jnp.float32)
        ra = lax.broadcasted_iota(jnp.int32, (800, 800), 0)
        rb = lax.broadcasted_iota(jnp.int32, (800, 800), 1)
        m2t = (rb < ra).astype(jnp.float32)
        rowtot = jnp.sum(tie, axis=1, keepdims=True)
        rowpref = jnp.dot(m2t, rowtot, preferred_element_type=jnp.float32)
        prefix = pref_in_row + rowpref
        sel = (tie > 0.0) & (prefix < needed)
        top_ref[...] = jnp.where((key > T) | sel, 1.0, 0.0)

    smem = pl.BlockSpec(memory_space=pltpu.SMEM)
    return pl.pallas_call(
        body,
        out_shape=(jax.ShapeDtypeStruct((800, 128), jnp.float32),
                   jax.ShapeDtypeStruct((800, 128), jnp.float32)),
        in_specs=[pl.BlockSpec(), pl.BlockSpec(), pl.BlockSpec(), smem, smem],
    )(p3a, p3b, z0, b3, karr)


def kernel(x, edge_attr, W1, b1, W2, b2, W3, b3, edge_index, k):
    f32, i32 = jnp.float32, jnp.int32
    xf = x.reshape(-1).astype(f32)
    n_extra = E_PAD - E_EDGES
    pad_idx = (N_NODES + (jnp.arange(n_extra, dtype=i32) % 2048)).astype(i32)
    rowp = jnp.concatenate([edge_index[0].astype(i32), pad_idx]).reshape(ROWS_E, 128)
    colp = jnp.concatenate([edge_index[1].astype(i32), pad_idx]).reshape(ROWS_E, 128)
    ewp = jnp.concatenate([edge_attr.astype(f32),
                           jnp.zeros((n_extra,), f32)]).reshape(ROWS_E, 128)
    xp = jnp.pad(xf, (0, N_PAD - N_NODES)).reshape(1, N_PAD)
    zeros1 = jnp.zeros((1, N_PAD), f32)
    zeros8 = jnp.zeros((8, N_PAD), f32)

    deg_k = _make_deg_kernel()
    norm_k = _make_norm_kernel()
    prop1_1 = _make_prop_kernel(1, 1)
    prop1_2 = _make_prop_kernel(1, 2)
    prop1_3 = _make_prop_kernel(1, 3)
    prop8_1 = _make_prop_kernel(8, 1)
    prop8_2 = _make_prop_kernel(8, 2)

    degp = deg_k(colp, ewp)
    dis2 = _tc_dis(degp[0].reshape(800, 128), degp[1].reshape(800, 128))
    norm2 = norm_k(dis2.reshape(N_PAD), rowp, colp, ewp)

    s1 = prop1_1(xp, rowp, colp, norm2)
    s2 = prop1_2(s1[0], s1[1], rowp, colp, norm2)
    s3 = prop1_2(s2[0], s2[1], rowp, colp, norm2)
    r = lambda a: a.reshape(800, 128)
    w1r = W1.reshape(4, 8).astype(f32)
    h1 = _tc_layer1(r(xp), r(s1[0]), r(s1[1]), r(s2[0]), r(s2[1]),
                    r(s3[0]), r(s3[1]), w1r, b1.reshape(1, 8).astype(f32))

    h1p = h1.reshape(8, N_PAD)
    g1 = prop8_1(h1p, rowp, colp, norm2)
    g2 = prop8_2(g1[0], g1[1], rowp, colp, norm2)
    g3 = prop8_2(g2[0], g2[1], rowp, colp, norm2)
    w2r = W2.reshape(32, 8).astype(f32)
    w3r = W3.reshape(4, 8).astype(f32)
    rr = lambda a: a.reshape(8, 800, 128)
    z = _tc_layer2(rr(h1p), rr(g1[0]), rr(g1[1]), rr(g2[0]), rr(g2[1]),
                   rr(g3[0]), rr(g3[1]), w2r, b2.reshape(1, 8).astype(f32), w3r)
    z = z.reshape(4, N_PAD)

    p1 = prop1_1(z[3].reshape(1, N_PAD), rowp, colp, norm2)
    p2 = prop1_3(p1[0], p1[1], z[2].reshape(1, N_PAD), rowp, colp, norm2)
    p3 = prop1_3(p2[0], p2[1], z[1].reshape(1, N_PAD), rowp, colp, norm2)

    karr = jnp.reshape(k, (1, 1)).astype(i32)
    xs2, top2 = _tc_final(r(p3[0]), r(p3[1]), z[0].reshape(800, 128),
                          b3.reshape(1, 1).astype(f32), karr)
    xs_flat = xs2.reshape(-1)[:N_NODES]
    top_flat = top2.reshape(-1)[:N_NODES]
    return jnp.column_stack((xs_flat, top_flat))


# speedup vs baseline: 69.1928x; 2.5049x over previous
import functools

import jax
import jax.numpy as jnp
from jax import lax
from jax.experimental import pallas as pl
from jax.experimental.pallas import tpu as pltpu
from jax.experimental.pallas import tpu_sc as plsc

N_NODES = 100000
N_PAD = 102400
E_EDGES = 3200000
E_PAD = 3211264
ROWS_E = E_PAD // 128
NC, NS = 2, 16
NW = NC * NS
RW = ROWS_E // NW
U = 4
C_EDGES = U * 128
NCHUNK = RW // U
SLICE = N_PAD // NS
ROWS_ALLOC = ROWS_E + U

_mesh = lambda: plsc.VectorSubcoreMesh(
    core_axis_name="c", subcore_axis_name="s", num_cores=NC, num_subcores=NS)


def _vec_loop(n, body):
    lax.fori_loop(0, n, lambda i, _: (body(i), 0)[1], 0)


def _zero_buf1(buf, n):
    def b(i):
        buf[pl.ds(i * 16, 16)] = jnp.zeros((16,), jnp.float32)
    _vec_loop(n // 16, b)


def _make_deg_kernel():
    @functools.partial(
        pl.kernel,
        out_type=jax.ShapeDtypeStruct((NC, N_PAD), jnp.float32),
        mesh=_mesh(),
        scratch_types=[
            pltpu.VMEM_SHARED((N_PAD,), jnp.float32),
            pltpu.VMEM((SLICE,), jnp.float32),
            pltpu.VMEM((U, 128), jnp.int32),
            pltpu.VMEM((C_EDGES,), jnp.float32),
            pltpu.SemaphoreType.DMA,
        ],
    )
    def deg_kernel(col_hbm, ew_hbm, out_hbm, acc, zbuf, colv, ewv, sem):
        cid = lax.axis_index("c")
        sid = lax.axis_index("s")
        wid = cid * NS + sid
        base = sid * SLICE
        _zero_buf1(zbuf, SLICE)
        pltpu.sync_copy(zbuf, acc.at[pl.ds(base, SLICE)])
        plsc.subcore_barrier()
        erow0 = wid * RW

        def chunk(ci, _):
            r0 = erow0 + ci * U
            pltpu.sync_copy(col_hbm.at[pl.ds(r0, U)], colv)
            pltpu.sync_copy(ew_hbm.at[pl.ds(r0 * 128, C_EDGES)], ewv)

            def absb(i):
                s = pl.ds(i * 16, 16)
                ewv[s] = jnp.abs(ewv[s])
            _vec_loop(C_EDGES // 16, absb)
            ds = [pltpu.async_copy(ewv.at[pl.ds(j * 128, 128)],
                                   acc.at[colv.at[j]], sem, add=True)
                  for j in range(U)]
            for d in ds:
                d.wait()
            return 0

        lax.fori_loop(0, NCHUNK, chunk, 0)
        plsc.subcore_barrier()
        pltpu.sync_copy(acc.at[pl.ds(base, SLICE)],
                        out_hbm.at[cid, pl.ds(base, SLICE)])

    return deg_kernel


def _make_norm_kernel():
    @functools.partial(
        pl.kernel,
        out_type=jax.ShapeDtypeStruct((E_PAD + C_EDGES,), jnp.float32),
        mesh=_mesh(),
        scratch_types=[
            pltpu.VMEM_SHARED((N_PAD,), jnp.float32),
            pltpu.VMEM((SLICE,), jnp.float32),
            pltpu.VMEM((U, 128), jnp.int32),
            pltpu.VMEM((U, 128), jnp.int32),
            pltpu.VMEM((C_EDGES,), jnp.float32),
            pltpu.VMEM((C_EDGES,), jnp.float32),
            pltpu.VMEM((C_EDGES,), jnp.float32),
            pltpu.SemaphoreType.DMA,
        ],
    )
    def norm_kernel(dis_hbm, row_hbm, col_hbm, ew_hbm, out_hbm,
                    dis_sp, va, rowv, colv, ewv, gr, gc, sem):
        cid = lax.axis_index("c")
        sid = lax.axis_index("s")
        wid = cid * NS + sid
        base = sid * SLICE
        pltpu.sync_copy(dis_hbm.at[pl.ds(base, SLICE)], va)
        pltpu.sync_copy(va, dis_sp.at[pl.ds(base, SLICE)])
        plsc.subcore_barrier()
        erow0 = wid * RW

        def chunk(ci, _):
            r0 = erow0 + ci * U
            pltpu.sync_copy(row_hbm.at[pl.ds(r0, U)], rowv)
            pltpu.sync_copy(col_hbm.at[pl.ds(r0, U)], colv)
            pltpu.sync_copy(ew_hbm.at[pl.ds(r0 * 128, C_EDGES)], ewv)
            ds = []
            for j in range(U):
                ds.append(pltpu.async_copy(
                    dis_sp.at[rowv.at[j]], gr.at[pl.ds(j * 128, 128)], sem))
                ds.append(pltpu.async_copy(
                    dis_sp.at[colv.at[j]], gc.at[pl.ds(j * 128, 128)], sem))
            for d in ds:
                d.wait()

            def mulb(i):
                s = pl.ds(i * 16, 16)
                ewv[s] = gr[s] * jnp.abs(ewv[s]) * gc[s]
            _vec_loop(C_EDGES // 16, mulb)
            pltpu.sync_copy(ewv, out_hbm.at[pl.ds(r0 * 128, C_EDGES)])
            return 0

        lax.fori_loop(0, NCHUNK, chunk, 0)

    return norm_kernel


def _make_prop_kernel(P, n_in):
    scratch = (
        [pltpu.VMEM_SHARED((N_PAD,), jnp.float32) for _ in range(2 * P)]
        + [pltpu.VMEM((SLICE,), jnp.float32),
           pltpu.VMEM((SLICE,), jnp.float32),
           pltpu.VMEM((SLICE,), jnp.float32),
           pltpu.VMEM((U, 128), jnp.int32),
           pltpu.VMEM((U, 128), jnp.int32),
           pltpu.VMEM((C_EDGES,), jnp.float32),
           pltpu.VMEM((U, 128), jnp.int32),
           pltpu.VMEM((U, 128), jnp.int32),
           pltpu.VMEM((C_EDGES,), jnp.float32),
           pltpu.VMEM((P, U, 128), jnp.float32),
           pltpu.SemaphoreType.DMA,
           pltpu.SemaphoreType.DMA,
           pltpu.SemaphoreType.DMA]
    )

    @functools.partial(
        pl.kernel,
        out_type=jax.ShapeDtypeStruct((NC, P, N_PAD), jnp.float32),
        mesh=_mesh(),
        scratch_types=scratch,
    )
    def prop_kernel(*refs):
        ins = refs[:n_in]
        row_hbm, col_hbm, norm_hbm, out_hbm = refs[n_in:n_in + 4]
        sc = refs[n_in + 4:]
        h_sp = sc[:P]
        acc_sp = sc[P:2 * P]
        (va, vb, zbuf, rowvA, colvA, normvA, rowvB, colvB, normvB,
         gbuf, sem_e, sem_g, sem_s) = sc[2 * P:]

        cid = lax.axis_index("c")
        sid = lax.axis_index("s")
        wid = cid * NS + sid
        base = sid * SLICE
        _zero_buf1(zbuf, SLICE)
        for p in range(P):
            pltpu.sync_copy(ins[0].at[p, pl.ds(base, SLICE)], va)
            for a in range(1, n_in):
                pltpu.sync_copy(ins[a].at[p, pl.ds(base, SLICE)], vb)

                def addb(i):
                    s = pl.ds(i * 16, 16)
                    va[s] = va[s] + vb[s]
                _vec_loop(SLICE // 16, addb)
            pltpu.sync_copy(va, h_sp[p].at[pl.ds(base, SLICE)])
            pltpu.sync_copy(zbuf, acc_sp[p].at[pl.ds(base, SLICE)])
        plsc.subcore_barrier()
        erow0 = wid * RW

        def load_edges(r0, rowv, colv, normv):
            pltpu.async_copy(row_hbm.at[pl.ds(r0, U)], rowv, sem_e)
            pltpu.async_copy(col_hbm.at[pl.ds(r0, U)], colv, sem_e)
            pltpu.async_copy(norm_hbm.at[pl.ds(r0 * 128, C_EDGES)], normv,
                             sem_e)

        def wait_edges(rowv, colv, normv):
            pltpu.make_async_copy(row_hbm.at[pl.ds(0, U)], rowv, sem_e).wait()
            pltpu.make_async_copy(col_hbm.at[pl.ds(0, U)], colv, sem_e).wait()
            pltpu.make_async_copy(norm_hbm.at[pl.ds(0, C_EDGES)], normv,
                                  sem_e).wait()

        def fire_gathers(p, rowv):
            for j in range(U):
                pltpu.async_copy(h_sp[p].at[rowv.at[j]], gbuf.at[p, j], sem_g)

        def drain_gathers(p, rowv):
            for j in range(U):
                pltpu.make_async_copy(h_sp[p].at[rowv.at[j]], gbuf.at[p, j],
                                      sem_g).wait()

        def do_chunk(rowv, colv, normv):
            fire_gathers(0, rowv)
            for p in range(P):
                if p + 1 < P:
                    fire_gathers(p + 1, rowv)
                drain_gathers(p, rowv)
                for j in range(U):
                    for v in range(8):
                        s = pl.ds(v * 16, 16)
                        es = pl.ds(j * 128 + v * 16, 16)
                        gbuf[p, j, s] = gbuf[p, j, s] * normv[es]
                for j in range(U):
                    pltpu.async_copy(gbuf.at[p, j], acc_sp[p].at[colv.at[j]],
                                     sem_s, add=True)
            for p in range(P):
                for j in range(U):
                    pltpu.make_async_copy(gbuf.at[p, j],
                                          acc_sp[p].at[colv.at[j]],
                                          sem_s).wait()

        load_edges(erow0, rowvA, colvA, normvA)

        def pair(ci2, _):
            a0 = erow0 + (2 * ci2) * U
            wait_edges(rowvA, colvA, normvA)
            load_edges(a0 + U, rowvB, colvB, normvB)
            do_chunk(rowvA, colvA, normvA)
            wait_edges(rowvB, colvB, normvB)
            load_edges(a0 + 2 * U, rowvA, colvA, normvA)
            do_chunk(rowvB, colvB, normvB)
            return 0

        lax.fori_loop(0, NCHUNK // 2, pair, 0)
        wait_edges(rowvA, colvA, normvA)
        plsc.subcore_barrier()
        for p in range(P):
            pltpu.sync_copy(acc_sp[p].at[pl.ds(base, SLICE)],
                            out_hbm.at[cid, p, pl.ds(base, SLICE)])

    return prop_kernel


def _tc_dis(dega, degb):
    def body(a_ref, b_ref, o_ref):
        d = a_ref[...] + b_ref[...]
        o_ref[...] = jnp.where(d > 0.0, lax.rsqrt(d), 0.0)
    return pl.pallas_call(
        body, out_shape=jax.ShapeDtypeStruct((800, 128), jnp.float32),
    )(dega, degb)


def _tc_layer1(x2, s1a, s1b, s2a, s2b, s3a, s3b, w, b):
    def body(x_ref, a1, b1r, a2, b2r, a3, b3r, w_ref, bias_ref, o_ref):
        s0 = x_ref[...]
        s1 = a1[...] + b1r[...]
        s2 = a2[...] + b2r[...]
        s3 = a3[...] + b3r[...]
        planes = []
        for c in range(8):
            acc = (s0 * w_ref[0, c] + s1 * w_ref[1, c]
                   + s2 * w_ref[2, c] + s3 * w_ref[3, c] + bias_ref[0, c])
            planes.append(jnp.maximum(acc, 0.0))
        o_ref[...] = jnp.stack(planes, axis=0)
    smem = pl.BlockSpec(memory_space=pltpu.SMEM)
    return pl.pallas_call(
        body,
        out_shape=jax.ShapeDtypeStruct((8, 800, 128), jnp.float32),
        in_specs=[pl.BlockSpec()] * 7 + [smem, smem],
    )(x2, s1a, s1b, s2a, s2b, s3a, s3b, w, b)


def _tc_layer2(h1, g1a, g1b, g2a, g2b, g3a, g3b, w2, b2, w3):
    GB = 200

    def body(h_ref, a1, b1r, a2, b2r, a3, b3r, w2_ref, bias_ref, w3_ref, o_ref):
        bf = lambda v: v.astype(jnp.bfloat16).astype(jnp.float32)
        h = bf(h_ref[...])
        g1 = bf(a1[...] + b1r[...])
        g2 = bf(a2[...] + b2r[...])
        g3 = bf(a3[...] + b3r[...])
        h2 = []
        for c in range(8):
            acc = bias_ref[0, c]
            accv = jnp.full((GB, 128), acc, jnp.float32)
            for d in range(8):
                accv = accv + h[d] * bf(w2_ref[d, c])
                accv = accv + g1[d] * bf(w2_ref[8 + d, c])
                accv = accv + g2[d] * bf(w2_ref[16 + d, c])
                accv = accv + g3[d] * bf(w2_ref[24 + d, c])
            h2.append(bf(jnp.maximum(accv, 0.0)))
        zs = []
        for i in range(4):
            z = h2[0] * bf(w3_ref[i, 0])
            for c in range(1, 8):
                z = z + h2[c] * bf(w3_ref[i, c])
            zs.append(z)
        o_ref[...] = jnp.stack(zs, axis=0)

    smem = pl.BlockSpec(memory_space=pltpu.SMEM)
    big = pl.BlockSpec((8, GB, 128), lambda r: (0, r, 0))
    return pl.pallas_call(
        body,
        grid=(800 // GB,),
        out_shape=jax.ShapeDtypeStruct((4, 800, 128), jnp.float32),
        in_specs=[big] * 7 + [smem, smem, smem],
        out_specs=pl.BlockSpec((4, GB, 128), lambda r: (0, r, 0)),
    )(h1, g1a, g1b, g2a, g2b, g3a, g3b, w2, b2, w3)


def _tc_final(p3a, p3b, z0, b3, karr):
    def body(a_ref, b_ref, z_ref, b3_ref, k_ref, xs_ref, top_ref):
        INT_MIN = jnp.int32(-(2 ** 31))
        NF = jnp.float32(N_NODES)
        pre = a_ref[...] + b_ref[...] + z_ref[...] + b3_ref[0, 0]
        h3 = jnp.maximum(pre, 0.0)
        ri = lax.broadcasted_iota(jnp.int32, (800, 128), 0)
        ci = lax.broadcasted_iota(jnp.int32, (800, 128), 1)
        fidx = ri * 128 + ci
        mask = fidx < N_NODES
        h3m = jnp.where(mask, h3, 0.0)
        mean = jnp.sum(h3m) / NF
        dev = jnp.where(mask, h3 - mean, 0.0)
        var = jnp.sum(dev * dev) / NF
        xs = (h3 - mean) * lax.rsqrt(var + 1e-5)
        xs_ref[...] = xs
        u = lax.bitcast_convert_type(xs, jnp.int32)
        key = u ^ (jnp.right_shift(u, 31) & jnp.int32(0x7FFFFFFF))
        key = jnp.where(mask, key, INT_MIN)
        kf = k_ref[0, 0].astype(jnp.float32)
        cnt0 = jnp.sum(jnp.where(key >= 0, 1.0, 0.0))
        T = jnp.where(cnt0 >= kf, jnp.int32(0), INT_MIN)
        for bit in range(30, -1, -1):
            cand = T + jnp.int32(1 << bit)
            cnt = jnp.sum(jnp.where(key >= cand, 1.0, 0.0))
            T = jnp.where(cnt >= kf, cand, T)
        cnt_gt = jnp.sum(jnp.where(key > T, 1.0, 0.0))
        needed = kf - cnt_gt
        tie = jnp.where(key == T, 1.0, 0.0)
        ia = lax.broadcasted_iota(jnp.int32, (128, 128), 0)
        ib = lax.broadcasted_iota(jnp.int32, (128, 128), 1)
        m = (ia < ib).astype(jnp.float32)
        pref_in_row = jnp.dot(tie, m, preferred_element_type=jnp.float32)
        ra = lax.broadcasted_iota(jnp.int32, (800, 800), 0)
        rb = lax.broadcasted_iota(jnp.int32, (800, 800), 1)
        m2t = (rb < ra).astype(jnp.float32)
        rowtot = jnp.sum(tie, axis=1, keepdims=True)
        rowpref = jnp.dot(m2t, rowtot, preferred_element_type=jnp.float32)
        prefix = pref_in_row + rowpref
        sel = (tie > 0.0) & (prefix < needed)
        top_ref[...] = jnp.where((key > T) | sel, 1.0, 0.0)

    smem = pl.BlockSpec(memory_space=pltpu.SMEM)
    return pl.pallas_call(
        body,
        out_shape=(jax.ShapeDtypeStruct((800, 128), jnp.float32),
                   jax.ShapeDtypeStruct((800, 128), jnp.float32)),
        in_specs=[pl.BlockSpec(), pl.BlockSpec(), pl.BlockSpec(), smem, smem],
    )(p3a, p3b, z0, b3, karr)


def kernel(x, edge_attr, W1, b1, W2, b2, W3, b3, edge_index, k):
    f32, i32 = jnp.float32, jnp.int32
    xf = x.reshape(-1).astype(f32)
    n_extra = ROWS_ALLOC * 128 - E_EDGES
    pad_idx = (N_NODES + (jnp.arange(n_extra, dtype=i32) % 2048)).astype(i32)
    rowp = jnp.concatenate([edge_index[0].astype(i32),
                            pad_idx]).reshape(ROWS_ALLOC, 128)
    colp = jnp.concatenate([edge_index[1].astype(i32),
                            pad_idx]).reshape(ROWS_ALLOC, 128)
    ewp = jnp.concatenate([edge_attr.astype(f32),
                           jnp.zeros((E_PAD - E_EDGES,), f32)])
    xp = jnp.pad(xf, (0, N_PAD - N_NODES)).reshape(1, N_PAD)
    zeros1 = jnp.zeros((1, N_PAD), f32)

    deg_k = _make_deg_kernel()
    norm_k = _make_norm_kernel()
    prop1_1 = _make_prop_kernel(1, 1)
    prop1_2 = _make_prop_kernel(1, 2)
    prop1_3 = _make_prop_kernel(1, 3)
    prop8_1 = _make_prop_kernel(8, 1)
    prop8_2 = _make_prop_kernel(8, 2)
    del zeros1

    degp = deg_k(colp[:ROWS_E], ewp)
    dis2 = _tc_dis(degp[0].reshape(800, 128), degp[1].reshape(800, 128))
    normf = norm_k(dis2.reshape(N_PAD), rowp[:ROWS_E], colp[:ROWS_E], ewp)

    s1 = prop1_1(xp, rowp, colp, normf)
    s2 = prop1_2(s1[0], s1[1], rowp, colp, normf)
    s3 = prop1_2(s2[0], s2[1], rowp, colp, normf)
    r = lambda a: a.reshape(800, 128)
    w1r = W1.reshape(4, 8).astype(f32)
    h1 = _tc_layer1(r(xp), r(s1[0]), r(s1[1]), r(s2[0]), r(s2[1]),
                    r(s3[0]), r(s3[1]), w1r, b1.reshape(1, 8).astype(f32))

    h1p = h1.reshape(8, N_PAD)
    g1 = prop8_1(h1p, rowp, colp, normf)
    g2 = prop8_2(g1[0], g1[1], rowp, colp, normf)
    g3 = prop8_2(g2[0], g2[1], rowp, colp, normf)
    w2r = W2.reshape(32, 8).astype(f32)
    w3r = W3.reshape(4, 8).astype(f32)
    rr = lambda a: a.reshape(8, 800, 128)
    z = _tc_layer2(rr(h1p), rr(g1[0]), rr(g1[1]), rr(g2[0]), rr(g2[1]),
                   rr(g3[0]), rr(g3[1]), w2r, b2.reshape(1, 8).astype(f32), w3r)
    z = z.reshape(4, N_PAD)

    p1 = prop1_1(z[3].reshape(1, N_PAD), rowp, colp, normf)
    p2 = prop1_3(p1[0], p1[1], z[2].reshape(1, N_PAD), rowp, colp, normf)
    p3 = prop1_3(p2[0], p2[1], z[1].reshape(1, N_PAD), rowp, colp, normf)

    karr = jnp.reshape(k, (1, 1)).astype(i32)
    xs2, top2 = _tc_final(r(p3[0]), r(p3[1]), z[0].reshape(800, 128),
                          b3.reshape(1, 1).astype(f32), karr)
    xs_flat = xs2.reshape(-1)[:N_NODES]
    top_flat = top2.reshape(-1)[:N_NODES]
    return jnp.column_stack((xs_flat, top_flat))


# speedup vs baseline: 69.1967x; 1.0001x over previous
import functools

import jax
import jax.numpy as jnp
from jax import lax
from jax.experimental import pallas as pl
from jax.experimental.pallas import tpu as pltpu
from jax.experimental.pallas import tpu_sc as plsc

N_NODES = 100000
N_PAD = 102400
E_EDGES = 3200000
E_PAD = 3211264
ROWS_E = E_PAD // 128
NC, NS = 2, 16
NW = NC * NS
RW = ROWS_E // NW
U = 4
C_EDGES = U * 128
NCHUNK = RW // U
SLICE = N_PAD // NS
ROWS_ALLOC = ROWS_E + U

_mesh = lambda: plsc.VectorSubcoreMesh(
    core_axis_name="c", subcore_axis_name="s", num_cores=NC, num_subcores=NS)


def _vec_loop(n, body):
    lax.fori_loop(0, n, lambda i, _: (body(i), 0)[1], 0)


def _zero_buf1(buf, n):
    def b(i):
        buf[pl.ds(i * 16, 16)] = jnp.zeros((16,), jnp.float32)
    _vec_loop(n // 16, b)


def _make_deg_kernel():
    @functools.partial(
        pl.kernel,
        out_type=jax.ShapeDtypeStruct((NC, N_PAD), jnp.float32),
        mesh=_mesh(),
        scratch_types=[
            pltpu.VMEM_SHARED((N_PAD,), jnp.float32),
            pltpu.VMEM((SLICE,), jnp.float32),
            pltpu.VMEM((U, 128), jnp.int32),
            pltpu.VMEM((C_EDGES,), jnp.float32),
            pltpu.SemaphoreType.DMA,
        ],
    )
    def deg_kernel(col_hbm, ew_hbm, out_hbm, acc, zbuf, colv, ewv, sem):
        cid = lax.axis_index("c")
        sid = lax.axis_index("s")
        wid = cid * NS + sid
        base = sid * SLICE
        _zero_buf1(zbuf, SLICE)
        pltpu.sync_copy(zbuf, acc.at[pl.ds(base, SLICE)])
        plsc.subcore_barrier()
        erow0 = wid * RW

        def chunk(ci, _):
            r0 = erow0 + ci * U
            pltpu.sync_copy(col_hbm.at[pl.ds(r0, U)], colv)
            pltpu.sync_copy(ew_hbm.at[pl.ds(r0 * 128, C_EDGES)], ewv)

            def absb(i):
                s = pl.ds(i * 16, 16)
                ewv[s] = jnp.abs(ewv[s])
            _vec_loop(C_EDGES // 16, absb)
            ds = [pltpu.async_copy(ewv.at[pl.ds(j * 128, 128)],
                                   acc.at[colv.at[j]], sem, add=True)
                  for j in range(U)]
            for d in ds:
                d.wait()
            return 0

        lax.fori_loop(0, NCHUNK, chunk, 0)
        plsc.subcore_barrier()
        pltpu.sync_copy(acc.at[pl.ds(base, SLICE)],
                        out_hbm.at[cid, pl.ds(base, SLICE)])

    return deg_kernel


def _make_norm_kernel():
    @functools.partial(
        pl.kernel,
        out_type=jax.ShapeDtypeStruct((E_PAD + C_EDGES,), jnp.float32),
        mesh=_mesh(),
        scratch_types=[
            pltpu.VMEM_SHARED((N_PAD,), jnp.float32),
            pltpu.VMEM((SLICE,), jnp.float32),
            pltpu.VMEM((U, 128), jnp.int32),
            pltpu.VMEM((U, 128), jnp.int32),
            pltpu.VMEM((C_EDGES,), jnp.float32),
            pltpu.VMEM((C_EDGES,), jnp.float32),
            pltpu.VMEM((C_EDGES,), jnp.float32),
            pltpu.SemaphoreType.DMA,
        ],
    )
    def norm_kernel(dis_hbm, row_hbm, col_hbm, ew_hbm, out_hbm,
                    dis_sp, va, rowv, colv, ewv, gr, gc, sem):
        cid = lax.axis_index("c")
        sid = lax.axis_index("s")
        wid = cid * NS + sid
        base = sid * SLICE
        pltpu.sync_copy(dis_hbm.at[pl.ds(base, SLICE)], va)
        pltpu.sync_copy(va, dis_sp.at[pl.ds(base, SLICE)])
        plsc.subcore_barrier()
        erow0 = wid * RW

        def chunk(ci, _):
            r0 = erow0 + ci * U
            pltpu.sync_copy(row_hbm.at[pl.ds(r0, U)], rowv)
            pltpu.sync_copy(col_hbm.at[pl.ds(r0, U)], colv)
            pltpu.sync_copy(ew_hbm.at[pl.ds(r0 * 128, C_EDGES)], ewv)
            ds = []
            for j in range(U):
                ds.append(pltpu.async_copy(
                    dis_sp.at[rowv.at[j]], gr.at[pl.ds(j * 128, 128)], sem))
                ds.append(pltpu.async_copy(
                    dis_sp.at[colv.at[j]], gc.at[pl.ds(j * 128, 128)], sem))
            for d in ds:
                d.wait()

            def mulb(i):
                s = pl.ds(i * 16, 16)
                ewv[s] = gr[s] * jnp.abs(ewv[s]) * gc[s]
            _vec_loop(C_EDGES // 16, mulb)
            pltpu.sync_copy(ewv, out_hbm.at[pl.ds(r0 * 128, C_EDGES)])
            return 0

        lax.fori_loop(0, NCHUNK, chunk, 0)

    return norm_kernel


def _make_norm_kernel_tile_unused():
    @functools.partial(
        pl.kernel,
        out_type=jax.ShapeDtypeStruct((E_PAD + C_EDGES,), jnp.float32),
        mesh=_mesh(),
        compiler_params=pltpu.CompilerParams(needs_layout_passes=False),
        scratch_types=[
            pltpu.VMEM((N_PAD,), jnp.float32),
            pltpu.VMEM((U, 128), jnp.int32),
            pltpu.VMEM((U, 128), jnp.int32),
            pltpu.VMEM((C_EDGES,), jnp.float32),
            pltpu.VMEM((U, 128), jnp.int32),
            pltpu.VMEM((U, 128), jnp.int32),
            pltpu.VMEM((C_EDGES,), jnp.float32),
            pltpu.SemaphoreType.DMA,
        ],
    )
    def norm_kernel(dis_hbm, row_hbm, col_hbm, ew_hbm, out_hbm,
                    dis_t, rowvA, colvA, ewvA, rowvB, colvB, ewvB, sem_e):
        cid = lax.axis_index("c")
        sid = lax.axis_index("s")
        wid = cid * NS + sid
        pltpu.sync_copy(dis_hbm, dis_t)
        erow0 = wid * RW

        def load_edges(r0, rowv, colv, ewv):
            pltpu.async_copy(row_hbm.at[pl.ds(r0, U)], rowv, sem_e)
            pltpu.async_copy(col_hbm.at[pl.ds(r0, U)], colv, sem_e)
            pltpu.async_copy(ew_hbm.at[pl.ds(r0 * 128, C_EDGES)], ewv, sem_e)

        def wait_edges(rowv, colv, ewv):
            pltpu.make_async_copy(row_hbm.at[pl.ds(0, U)], rowv, sem_e).wait()
            pltpu.make_async_copy(col_hbm.at[pl.ds(0, U)], colv, sem_e).wait()
            pltpu.make_async_copy(ew_hbm.at[pl.ds(0, C_EDGES)], ewv,
                                  sem_e).wait()

        def do_chunk(r0, rowv, colv, ewv):
            for j in range(U):
                for v in range(8):
                    s16 = pl.ds(v * 16, 16)
                    es = pl.ds(j * 128 + v * 16, 16)
                    dr = plsc.load_gather(dis_t, [rowv[j, s16]])
                    dc = plsc.load_gather(dis_t, [colv[j, s16]])
                    ewv[es] = dr * jnp.abs(ewv[es]) * dc
            pltpu.sync_copy(ewv, out_hbm.at[pl.ds(r0 * 128, C_EDGES)])

        load_edges(erow0, rowvA, colvA, ewvA)

        def pair(ci2, _):
            a0 = erow0 + (2 * ci2) * U
            wait_edges(rowvA, colvA, ewvA)
            load_edges(a0 + U, rowvB, colvB, ewvB)
            do_chunk(a0, rowvA, colvA, ewvA)
            wait_edges(rowvB, colvB, ewvB)
            load_edges(a0 + 2 * U, rowvA, colvA, ewvA)
            do_chunk(a0 + U, rowvB, colvB, ewvB)
            return 0

        lax.fori_loop(0, NCHUNK // 2, pair, 0)
        wait_edges(rowvA, colvA, ewvA)

    return norm_kernel


def _make_prop1_tile_kernel(n_in):
    scratch = [
        pltpu.VMEM_SHARED((N_PAD,), jnp.float32),
        pltpu.VMEM((N_PAD,), jnp.float32),
        pltpu.VMEM((SLICE,), jnp.float32),
        pltpu.VMEM((U, 128), jnp.int32),
        pltpu.VMEM((U, 128), jnp.int32),
        pltpu.VMEM((C_EDGES,), jnp.float32),
        pltpu.VMEM((U, 128), jnp.int32),
        pltpu.VMEM((U, 128), jnp.int32),
        pltpu.VMEM((C_EDGES,), jnp.float32),
        pltpu.VMEM((C_EDGES,), jnp.float32),
        pltpu.VMEM((C_EDGES,), jnp.float32),
        pltpu.SemaphoreType.DMA,
        pltpu.SemaphoreType.DMA,
    ]

    @functools.partial(
        pl.kernel,
        out_type=jax.ShapeDtypeStruct((NC, 1, N_PAD), jnp.float32),
        mesh=_mesh(),
        compiler_params=pltpu.CompilerParams(needs_layout_passes=False),
        scratch_types=scratch,
    )
    def prop_kernel(*refs):
        ins = refs[:n_in]
        row_hbm, col_hbm, norm_hbm, out_hbm = refs[n_in:n_in + 4]
        (acc_sp, h_t, vb, rowvA, colvA, normvA, rowvB, colvB, normvB,
         gbufA, gbufB, sem_e, sem_s) = refs[n_in + 4:]

        cid = lax.axis_index("c")
        sid = lax.axis_index("s")
        wid = cid * NS + sid
        base = sid * SLICE
        pltpu.sync_copy(ins[0].at[0], h_t)
        for a in range(1, n_in):
            for seg in range(NS):
                sb = seg * SLICE
                pltpu.sync_copy(ins[a].at[0, pl.ds(sb, SLICE)], vb)

                def addb(i):
                    sa = pl.ds(sb + i * 16, 16)
                    h_t[sa] = h_t[sa] + vb[pl.ds(i * 16, 16)]
                _vec_loop(SLICE // 16, addb)
        _zero_buf1(vb, SLICE)
        pltpu.sync_copy(vb, acc_sp.at[pl.ds(base, SLICE)])
        plsc.subcore_barrier()
        erow0 = wid * RW

        def load_edges(r0, rowv, colv, normv):
            pltpu.async_copy(row_hbm.at[pl.ds(r0, U)], rowv, sem_e)
            pltpu.async_copy(col_hbm.at[pl.ds(r0, U)], colv, sem_e)
            pltpu.async_copy(norm_hbm.at[pl.ds(r0 * 128, C_EDGES)], normv,
                             sem_e)

        def wait_edges(rowv, colv, normv):
            pltpu.make_async_copy(row_hbm.at[pl.ds(0, U)], rowv, sem_e).wait()
            pltpu.make_async_copy(col_hbm.at[pl.ds(0, U)], colv, sem_e).wait()
            pltpu.make_async_copy(norm_hbm.at[pl.ds(0, C_EDGES)], normv,
                                  sem_e).wait()

        def compute(rowv, normv, gbuf):
            for j in range(U):
                for v in range(8):
                    s16 = pl.ds(v * 16, 16)
                    es = pl.ds(j * 128 + v * 16, 16)
                    vals = plsc.load_gather(h_t, [rowv[j, s16]])
                    gbuf[es] = vals * normv[es]

        def fire_scatters(colv, gbuf):
            for j in range(U):
                pltpu.async_copy(gbuf.at[pl.ds(j * 128, 128)],
                                 acc_sp.at[colv.at[j]], sem_s, add=True)

        def drain_scatters(colv, gbuf):
            for j in range(U):
                pltpu.make_async_copy(gbuf.at[pl.ds(j * 128, 128)],
                                      acc_sp.at[colv.at[j]], sem_s).wait()

        load_edges(erow0, rowvA, colvA, normvA)

        def pair(ci2, _):
            a0 = erow0 + (2 * ci2) * U
            wait_edges(rowvA, colvA, normvA)
            load_edges(a0 + U, rowvB, colvB, normvB)
            compute(rowvA, normvA, gbufA)
            fire_scatters(colvA, gbufA)
            wait_edges(rowvB, colvB, normvB)
            compute(rowvB, normvB, gbufB)
            drain_scatters(colvA, gbufA)
            load_edges(a0 + 2 * U, rowvA, colvA, normvA)
            fire_scatters(colvB, gbufB)
            drain_scatters(colvB, gbufB)
            return 0

        lax.fori_loop(0, NCHUNK // 2, pair, 0)
        wait_edges(rowvA, colvA, normvA)
        plsc.subcore_barrier()
        pltpu.sync_copy(acc_sp.at[pl.ds(base, SLICE)],
                        out_hbm.at[cid, 0, pl.ds(base, SLICE)])

    return prop_kernel


def _make_prop_kernel(P, n_in):
    scratch = (
        [pltpu.VMEM_SHARED((N_PAD,), jnp.float32) for _ in range(2 * P)]
        + [pltpu.VMEM((SLICE,), jnp.float32),
           pltpu.VMEM((SLICE,), jnp.float32),
           pltpu.VMEM((SLICE,), jnp.float32),
           pltpu.VMEM((U, 128), jnp.int32),
           pltpu.VMEM((U, 128), jnp.int32),
           pltpu.VMEM((C_EDGES,), jnp.float32),
           pltpu.VMEM((U, 128), jnp.int32),
           pltpu.VMEM((U, 128), jnp.int32),
           pltpu.VMEM((C_EDGES,), jnp.float32),
           pltpu.VMEM((P, U, 128), jnp.float32),
           pltpu.SemaphoreType.DMA,
           pltpu.SemaphoreType.DMA,
           pltpu.SemaphoreType.DMA]
    )

    @functools.partial(
        pl.kernel,
        out_type=jax.ShapeDtypeStruct((NC, P, N_PAD), jnp.float32),
        mesh=_mesh(),
        scratch_types=scratch,
    )
    def prop_kernel(*refs):
        ins = refs[:n_in]
        row_hbm, col_hbm, norm_hbm, out_hbm = refs[n_in:n_in + 4]
        sc = refs[n_in + 4:]
        h_sp = sc[:P]
        acc_sp = sc[P:2 * P]
        (va, vb, zbuf, rowvA, colvA, normvA, rowvB, colvB, normvB,
         gbuf, sem_e, sem_g, sem_s) = sc[2 * P:]

        cid = lax.axis_index("c")
        sid = lax.axis_index("s")
        wid = cid * NS + sid
        base = sid * SLICE
        _zero_buf1(zbuf, SLICE)
        for p in range(P):
            pltpu.sync_copy(ins[0].at[p, pl.ds(base, SLICE)], va)
            for a in range(1, n_in):
                pltpu.sync_copy(ins[a].at[p, pl.ds(base, SLICE)], vb)

                def addb(i):
                    s = pl.ds(i * 16, 16)
                    va[s] = va[s] + vb[s]
                _vec_loop(SLICE // 16, addb)
            pltpu.sync_copy(va, h_sp[p].at[pl.ds(base, SLICE)])
            pltpu.sync_copy(zbuf, acc_sp[p].at[pl.ds(base, SLICE)])
        plsc.subcore_barrier()
        erow0 = wid * RW

        def load_edges(r0, rowv, colv, normv):
            pltpu.async_copy(row_hbm.at[pl.ds(r0, U)], rowv, sem_e)
            pltpu.async_copy(col_hbm.at[pl.ds(r0, U)], colv, sem_e)
            pltpu.async_copy(norm_hbm.at[pl.ds(r0 * 128, C_EDGES)], normv,
                             sem_e)

        def wait_edges(rowv, colv, normv):
            pltpu.make_async_copy(row_hbm.at[pl.ds(0, U)], rowv, sem_e).wait()
            pltpu.make_async_copy(col_hbm.at[pl.ds(0, U)], colv, sem_e).wait()
            pltpu.make_async_copy(norm_hbm.at[pl.ds(0, C_EDGES)], normv,
                                  sem_e).wait()

        def fire_gathers(p, rowv):
            for j in range(U):
                pltpu.async_copy(h_sp[p].at[rowv.at[j]], gbuf.at[p, j], sem_g)

        def drain_gathers(p, rowv):
            for j in range(U):
                pltpu.make_async_copy(h_sp[p].at[rowv.at[j]], gbuf.at[p, j],
                                      sem_g).wait()

        def do_chunk(rowv, colv, normv):
            fire_gathers(0, rowv)
            for p in range(P):
                if p + 1 < P:
                    fire_gathers(p + 1, rowv)
                drain_gathers(p, rowv)
                for j in range(U):
                    for v in range(8):
                        s = pl.ds(v * 16, 16)
                        es = pl.ds(j * 128 + v * 16, 16)
                        gbuf[p, j, s] = gbuf[p, j, s] * normv[es]
                for j in range(U):
                    pltpu.async_copy(gbuf.at[p, j], acc_sp[p].at[colv.at[j]],
                                     sem_s, add=True)
            for p in range(P):
                for j in range(U):
                    pltpu.make_async_copy(gbuf.at[p, j],
                                          acc_sp[p].at[colv.at[j]],
                                          sem_s).wait()

        load_edges(erow0, rowvA, colvA, normvA)

        def pair(ci2, _):
            a0 = erow0 + (2 * ci2) * U
            wait_edges(rowvA, colvA, normvA)
            load_edges(a0 + U, rowvB, colvB, normvB)
            do_chunk(rowvA, colvA, normvA)
            wait_edges(rowvB, colvB, normvB)
            load_edges(a0 + 2 * U, rowvA, colvA, normvA)
            do_chunk(rowvB, colvB, normvB)
            return 0

        lax.fori_loop(0, NCHUNK // 2, pair, 0)
        wait_edges(rowvA, colvA, normvA)
        plsc.subcore_barrier()
        for p in range(P):
            pltpu.sync_copy(acc_sp[p].at[pl.ds(base, SLICE)],
                            out_hbm.at[cid, p, pl.ds(base, SLICE)])

    return prop_kernel


def _tc_dis(dega, degb):
    def body(a_ref, b_ref, o_ref):
        d = a_ref[...] + b_ref[...]
        o_ref[...] = jnp.where(d > 0.0, lax.rsqrt(d), 0.0)
    return pl.pallas_call(
        body, out_shape=jax.ShapeDtypeStruct((800, 128), jnp.float32),
    )(dega, degb)


def _tc_layer1(x2, s1a, s1b, s2a, s2b, s3a, s3b, w, b):
    def body(x_ref, a1, b1r, a2, b2r, a3, b3r, w_ref, bias_ref, o_ref):
        s0 = x_ref[...]
        s1 = a1[...] + b1r[...]
        s2 = a2[...] + b2r[...]
        s3 = a3[...] + b3r[...]
        planes = []
        for c in range(8):
            acc = (s0 * w_ref[0, c] + s1 * w_ref[1, c]
                   + s2 * w_ref[2, c] + s3 * w_ref[3, c] + bias_ref[0, c])
            planes.append(jnp.maximum(acc, 0.0))
        o_ref[...] = jnp.stack(planes, axis=0)
    smem = pl.BlockSpec(memory_space=pltpu.SMEM)
    return pl.pallas_call(
        body,
        out_shape=jax.ShapeDtypeStruct((8, 800, 128), jnp.float32),
        in_specs=[pl.BlockSpec()] * 7 + [smem, smem],
    )(x2, s1a, s1b, s2a, s2b, s3a, s3b, w, b)


def _tc_layer2(h1, g1a, g1b, g2a, g2b, g3a, g3b, w2, b2, w3):
    GB = 200

    def body(h_ref, a1, b1r, a2, b2r, a3, b3r, w2_ref, bias_ref, w3_ref, o_ref):
        bf = lambda v: v.astype(jnp.bfloat16).astype(jnp.float32)
        h = bf(h_ref[...])
        g1 = bf(a1[...] + b1r[...])
        g2 = bf(a2[...] + b2r[...])
        g3 = bf(a3[...] + b3r[...])
        h2 = []
        for c in range(8):
            acc = bias_ref[0, c]
            accv = jnp.full((GB, 128), acc, jnp.float32)
            for d in range(8):
                accv = accv + h[d] * bf(w2_ref[d, c])
                accv = accv + g1[d] * bf(w2_ref[8 + d, c])
                accv = accv + g2[d] * bf(w2_ref[16 + d, c])
                accv = accv + g3[d] * bf(w2_ref[24 + d, c])
            h2.append(bf(jnp.maximum(accv, 0.0)))
        zs = []
        for i in range(4):
            z = h2[0] * bf(w3_ref[i, 0])
            for c in range(1, 8):
                z = z + h2[c] * bf(w3_ref[i, c])
            zs.append(z)
        o_ref[...] = jnp.stack(zs, axis=0)

    smem = pl.BlockSpec(memory_space=pltpu.SMEM)
    big = pl.BlockSpec((8, GB, 128), lambda r: (0, r, 0))
    return pl.pallas_call(
        body,
        grid=(800 // GB,),
        out_shape=jax.ShapeDtypeStruct((4, 800, 128), jnp.float32),
        in_specs=[big] * 7 + [smem, smem, smem],
        out_specs=pl.BlockSpec((4, GB, 128), lambda r: (0, r, 0)),
    )(h1, g1a, g1b, g2a, g2b, g3a, g3b, w2, b2, w3)


def _tc_final(p3a, p3b, z0, b3, karr):
    def body(a_ref, b_ref, z_ref, b3_ref, k_ref, xs_ref, top_ref):
        INT_MIN = jnp.int32(-(2 ** 31))
        NF = jnp.float32(N_NODES)
        pre = a_ref[...] + b_ref[...] + z_ref[...] + b3_ref[0, 0]
        h3 = jnp.maximum(pre, 0.0)
        ri = lax.broadcasted_iota(jnp.int32, (800, 128), 0)
        ci = lax.broadcasted_iota(jnp.int32, (800, 128), 1)
        fidx = ri * 128 + ci
        mask = fidx < N_NODES
        h3m = jnp.where(mask, h3, 0.0)
        mean = jnp.sum(h3m) / NF
        dev = jnp.where(mask, h3 - mean, 0.0)
        var = jnp.sum(dev * dev) / NF
        xs = (h3 - mean) * lax.rsqrt(var + 1e-5)
        xs_ref[...] = xs
        u = lax.bitcast_convert_type(xs, jnp.int32)
        key = u ^ (jnp.right_shift(u, 31) & jnp.int32(0x7FFFFFFF))
        key = jnp.where(mask, key, INT_MIN)
        kf = k_ref[0, 0].astype(jnp.float32)
        cnt0 = jnp.sum(jnp.where(key >= 0, 1.0, 0.0))
        T = jnp.where(cnt0 >= kf, jnp.int32(0), INT_MIN)
        for bit in range(30, -1, -1):
            cand = T + jnp.int32(1 << bit)
            cnt = jnp.sum(jnp.where(key >= cand, 1.0, 0.0))
            T = jnp.where(cnt >= kf, cand, T)
        cnt_gt = jnp.sum(jnp.where(key > T, 1.0, 0.0))
        needed = kf - cnt_gt
        tie = jnp.where(key == T, 1.0, 0.0)
        ia = lax.broadcasted_iota(jnp.int32, (128, 128), 0)
        ib = lax.broadcasted_iota(jnp.int32, (128, 128), 1)
        m = (ia < ib).astype(jnp.float32)
        pref_in_row = jnp.dot(tie, m, preferred_element_type=jnp.float32)
        ra = lax.broadcasted_iota(jnp.int32, (800, 800), 0)
        rb = lax.broadcasted_iota(jnp.int32, (800, 800), 1)
        m2t = (rb < ra).astype(jnp.float32)
        rowtot = jnp.sum(tie, axis=1, keepdims=True)
        rowpref = jnp.dot(m2t, rowtot, preferred_element_type=jnp.float32)
        prefix = pref_in_row + rowpref
        sel = (tie > 0.0) & (prefix < needed)
        top_ref[...] = jnp.where((key > T) | sel, 1.0, 0.0)

    smem = pl.BlockSpec(memory_space=pltpu.SMEM)
    return pl.pallas_call(
        body,
        out_shape=(jax.ShapeDtypeStruct((800, 128), jnp.float32),
                   jax.ShapeDtypeStruct((800, 128), jnp.float32)),
        in_specs=[pl.BlockSpec(), pl.BlockSpec(), pl.BlockSpec(), smem, smem],
    )(p3a, p3b, z0, b3, karr)


def kernel(x, edge_attr, W1, b1, W2, b2, W3, b3, edge_index, k):
    f32, i32 = jnp.float32, jnp.int32
    xf = x.reshape(-1).astype(f32)
    n_extra = ROWS_ALLOC * 128 - E_EDGES
    pad_idx = (N_NODES + (jnp.arange(n_extra, dtype=i32) % 2048)).astype(i32)
    rowp = jnp.concatenate([edge_index[0].astype(i32),
                            pad_idx]).reshape(ROWS_ALLOC, 128)
    colp = jnp.concatenate([edge_index[1].astype(i32),
                            pad_idx]).reshape(ROWS_ALLOC, 128)
    ewp = jnp.concatenate([edge_attr.astype(f32),
                           jnp.zeros((ROWS_ALLOC * 128 - E_EDGES,), f32)])
    xp = jnp.pad(xf, (0, N_PAD - N_NODES)).reshape(1, N_PAD)
    zeros1 = jnp.zeros((1, N_PAD), f32)

    deg_k = _make_deg_kernel()
    norm_k = _make_norm_kernel()
    prop1_1 = _make_prop_kernel(1, 1)
    prop1_2 = _make_prop_kernel(1, 2)
    prop1_3 = _make_prop_kernel(1, 3)
    prop8_1 = _make_prop_kernel(8, 1)
    prop8_2 = _make_prop_kernel(8, 2)
    del zeros1

    degp = deg_k(colp[:ROWS_E], ewp[:E_PAD])
    dis2 = _tc_dis(degp[0].reshape(800, 128), degp[1].reshape(800, 128))
    normf = norm_k(dis2.reshape(N_PAD), rowp[:ROWS_E], colp[:ROWS_E],
                   ewp[:E_PAD])

    s1 = prop1_1(xp, rowp, colp, normf)
    s2 = prop1_2(s1[0], s1[1], rowp, colp, normf)
    s3 = prop1_2(s2[0], s2[1], rowp, colp, normf)
    r = lambda a: a.reshape(800, 128)
    w1r = W1.reshape(4, 8).astype(f32)
    h1 = _tc_layer1(r(xp), r(s1[0]), r(s1[1]), r(s2[0]), r(s2[1]),
                    r(s3[0]), r(s3[1]), w1r, b1.reshape(1, 8).astype(f32))

    h1p = h1.reshape(8, N_PAD)
    g1 = prop8_1(h1p, rowp, colp, normf)
    g2 = prop8_2(g1[0], g1[1], rowp, colp, normf)
    g3 = prop8_2(g2[0], g2[1], rowp, colp, normf)
    w2r = W2.reshape(32, 8).astype(f32)
    w3r = W3.reshape(4, 8).astype(f32)
    rr = lambda a: a.reshape(8, 800, 128)
    z = _tc_layer2(rr(h1p), rr(g1[0]), rr(g1[1]), rr(g2[0]), rr(g2[1]),
                   rr(g3[0]), rr(g3[1]), w2r, b2.reshape(1, 8).astype(f32), w3r)
    z = z.reshape(4, N_PAD)

    p1 = prop1_1(z[3].reshape(1, N_PAD), rowp, colp, normf)
    p2 = prop1_3(p1[0], p1[1], z[2].reshape(1, N_PAD), rowp, colp, normf)
    p3 = prop1_3(p2[0], p2[1], z[1].reshape(1, N_PAD), rowp, colp, normf)

    karr = jnp.reshape(k, (1, 1)).astype(i32)
    xs2, top2 = _tc_final(r(p3[0]), r(p3[1]), z[0].reshape(800, 128),
                          b3.reshape(1, 1).astype(f32), karr)
    xs_flat = xs2.reshape(-1)[:N_NODES]
    top_flat = top2.reshape(-1)[:N_NODES]
    return jnp.column_stack((xs_flat, top_flat))
